# baseline (speedup 1.0000x reference)
import functools

import jax
import jax.numpy as jnp
from jax import lax
from jax.experimental import pallas as pl
from jax.experimental.pallas import tpu as pltpu
from jax.experimental.pallas import tpu_sc as plsc

B, S = 4, 2048
HIDDEN = 768
N_TOK = B * S
NC, NS = 2, 16
NW = NC * NS
TOK_PER_W = N_TOK // NW
CHUNK = 32
N_CHUNKS = TOK_PER_W // CHUNK
NBUF = 2

_sc_mesh = plsc.VectorSubcoreMesh(
    core_axis_name="c", subcore_axis_name="s", num_cores=NC, num_subcores=NS
)


@functools.partial(
    pl.kernel,
    out_type=jax.ShapeDtypeStruct((N_TOK, HIDDEN), jnp.bfloat16),
    mesh=_sc_mesh,
    compiler_params=pltpu.CompilerParams(
        use_tc_tiling_on_sc=False, needs_layout_passes=False
    ),
    scratch_types=[
        pltpu.VMEM((N_CHUNKS, CHUNK), jnp.int32),
        pltpu.VMEM((NBUF, CHUNK, HIDDEN), jnp.float32),
        pltpu.VMEM((NBUF, CHUNK, HIDDEN), jnp.bfloat16),
        pltpu.SemaphoreType.DMA,
        pltpu.SemaphoreType.DMA,
    ],
)
def _sc_gather(ids_hbm, table_hbm, out_hbm, idx_v, rows_v, bf_v, gsem, wsem):
    wid = lax.axis_index("s") * NC + lax.axis_index("c")
    base = wid * TOK_PER_W
    pltpu.sync_copy(ids_hbm.at[wid], idx_v)
    ie = jnp.arange(16, dtype=jnp.int32) * 2
    io = ie + 1

    def gather(c):
        return pltpu.make_async_copy(
            table_hbm.at[idx_v.at[c]], rows_v.at[c % NBUF], gsem
        )

    def writeback(c):
        return pltpu.make_async_copy(
            bf_v.at[c % NBUF], out_hbm.at[pl.ds(base + c * CHUNK, CHUNK)], wsem
        )

    def convert(c):
        rows = rows_v.at[c % NBUF]
        bf = bf_v.at[c % NBUF]

        def tok_body(t, carry):
            row = jnp.full((16,), t, dtype=jnp.int32)
            for g in range(HIDDEN // 32):
                d = g * 32
                a = plsc.load_gather(rows, [row, ie + d])
                b = plsc.load_gather(rows, [row, io + d])
                bf[t, pl.ds(d, 32)] = plsc.pack(
                    a, b, format=plsc.PackFormat.INTERLEAVED
                )
            return carry

        lax.fori_loop(0, CHUNK, tok_body, 0)

    for c in range(min(NBUF, N_CHUNKS)):
        gather(c).start()
    for c in range(N_CHUNKS):
        gather(c).wait()
        if c >= NBUF:
            writeback(c - NBUF).wait()
        convert(c)
        writeback(c).start()
        if c + NBUF < N_CHUNKS:
            gather(c + NBUF).start()
    for c in range(max(0, N_CHUNKS - NBUF), N_CHUNKS):
        writeback(c).wait()


TC_BLK = 2048
POS_BLKS = S // TC_BLK


def _tc_body(g_ref, pos_ref, tt_ref, seg_ref, gam_ref, bet_ref, out_ref):
    x = g_ref[...].astype(jnp.float32) + pos_ref[...]
    tt = tt_ref[...]
    seg = seg_ref[...]
    s0 = seg[0:1, :]
    x = x + s0 + tt * (seg[1:2, :] - s0)
    mean = jnp.mean(x, axis=-1, keepdims=True)
    xc = x - mean
    var = jnp.mean(xc * xc, axis=-1, keepdims=True)
    y = xc * lax.rsqrt(var + 1e-12)
    out_ref[...] = y * gam_ref[...] + bet_ref[...]


def _tc_ln(gathered, pos_table, ttf, seg_table, gamma2d, beta2d):
    return pl.pallas_call(
        _tc_body,
        grid=(POS_BLKS, B),
        in_specs=[
            pl.BlockSpec((TC_BLK, HIDDEN), lambda j, b: (b * POS_BLKS + j, 0)),
            pl.BlockSpec((TC_BLK, HIDDEN), lambda j, b: (j, 0)),
            pl.BlockSpec((TC_BLK, 1), lambda j, b: (b * POS_BLKS + j, 0)),
            pl.BlockSpec((2, HIDDEN), lambda j, b: (0, 0)),
            pl.BlockSpec((1, HIDDEN), lambda j, b: (0, 0)),
            pl.BlockSpec((1, HIDDEN), lambda j, b: (0, 0)),
        ],
        out_specs=pl.BlockSpec((TC_BLK, HIDDEN), lambda j, b: (b * POS_BLKS + j, 0)),
        out_shape=jax.ShapeDtypeStruct((N_TOK, HIDDEN), jnp.float32),
    )(gathered, pos_table, ttf, seg_table, gamma2d, beta2d)


def kernel(input_ids, token_type_ids, token_table, pos_table, seg_table, ln_gamma, ln_beta):
    ids_w = input_ids.reshape(NW, N_CHUNKS, CHUNK).astype(jnp.int32)
    ttf = token_type_ids.reshape(N_TOK, 1).astype(jnp.float32)
    gathered = _sc_gather(ids_w, token_table)
    out = _tc_ln(
        gathered,
        pos_table,
        ttf,
        seg_table,
        ln_gamma.reshape(1, HIDDEN),
        ln_beta.reshape(1, HIDDEN),
    )
    return out.reshape(B, S, HIDDEN)
